# baseline (speedup 1.0000x reference)
import functools

import jax
import jax.numpy as jnp
from jax import lax
from jax.experimental import pallas as pl
from jax.experimental.pallas import tpu as pltpu
from jax.experimental.pallas import tpu_sc as plsc

N = 10000
E = 320000
D = 128

BN = 2000
BE = 2560

NC = 2
NS = 16
NW = NC * NS
GRP = 128
G = E // GRP
NPAD = 10240
RPT = NPAD // NS
JUNK_ROW = NPAD - 1

CHUNKS = ((0, 600), (600, 600), (1200, 600), (1800, 700))
GP2 = 2512
OCT_CHUNKS = ((0, 75), (75, 75), (150, 75), (225, 88))


def _silu(v):
    return v * jax.nn.sigmoid(v)


def _node_pre_body(x_ref, wa_ref, wb_ref, bm_ref, xa_ref, xb_ref):
    x = x_ref[...]
    xa_ref[...] = x @ wa_ref[...] + bm_ref[...]
    xb_ref[...] = x @ wb_ref[...]


def _edge_body(ea_ref, s_ref, we1_ref, be1_ref, wem_ref, wge_ref,
               bg_ref, wm2_ref, bm2_ref, msg_ref):
    h_e = _silu(ea_ref[...] @ we1_ref[...] + be1_ref[...])
    gate = jax.nn.sigmoid(h_e @ wge_ref[...] + bg_ref[...])
    pre = h_e @ wem_ref[...] + s_ref[...]
    msg_ref[...] = (_silu(pre) @ wm2_ref[...] + bm2_ref[...]) * gate


def _node_upd_body(x_ref, p0_ref, p1_ref, p2_ref, p3_ref, p4_ref, p5_ref,
                   p6_ref, p7_ref, wua_ref, wub_ref,
                   bu1_ref, wu2_ref, bu2_ref, g_ref, b_ref, out_ref):
    x = x_ref[...]
    aggr = ((p0_ref[...] + p1_ref[...]) + (p2_ref[...] + p3_ref[...])
            + (p4_ref[...] + p5_ref[...]) + (p6_ref[...] + p7_ref[...]))
    u1 = _silu(aggr @ wua_ref[...] + x @ wub_ref[...] + bu1_ref[...])
    upd = u1 @ wu2_ref[...] + bu2_ref[...]
    h = x + upd
    mu = jnp.mean(h, axis=-1, keepdims=True)
    var = jnp.mean((h - mu) ** 2, axis=-1, keepdims=True)
    out_ref[...] = (h - mu) * jax.lax.rsqrt(var + 1e-5) * g_ref[...] + b_ref[...]


def _full(shape):
    return pl.BlockSpec(shape, lambda i: tuple(0 for _ in shape))


def _node_pre(x, wa, wb, bm):
    return pl.pallas_call(
        _node_pre_body,
        grid=(N // BN,),
        in_specs=[
            pl.BlockSpec((BN, D), lambda i: (i, 0)),
            _full((D, D)), _full((D, D)), _full((1, D)),
        ],
        out_specs=[pl.BlockSpec((BN, D), lambda i: (i, 0)),
                   pl.BlockSpec((BN, D), lambda i: (i, 0))],
        out_shape=[jax.ShapeDtypeStruct((N, D), jnp.float32),
                   jax.ShapeDtypeStruct((N, D), jnp.float32)],
    )(x, wa, wb, bm)


def _edge_mlp(ea, s_k, we1, be1, wem, wge, bg2, wm2, bm2, e0):
    R = ea.shape[-1]
    ek = s_k.shape[0]
    blk0 = e0 // BE
    return pl.pallas_call(
        _edge_body,
        grid=(ek // BE,),
        in_specs=[
            pl.BlockSpec((BE, R), lambda i: (i + blk0, 0)),
            pl.BlockSpec((BE, D), lambda i: (i, 0)),
            _full((R, D)), _full((1, D)), _full((D, D)), _full((D, D)),
            _full((1, D)), _full((D, D)), _full((1, D)),
        ],
        out_specs=pl.BlockSpec((BE, D), lambda i: (i, 0)),
        out_shape=jax.ShapeDtypeStruct((ek, D), jnp.float32),
    )(ea, s_k, we1, be1, wem, wge, bg2, wm2, bm2)


def _node_upd(x, ps, wua, wub, bu1, wu2, bu2, g, b):
    return pl.pallas_call(
        _node_upd_body,
        grid=(N // BN,),
        in_specs=[pl.BlockSpec((BN, D), lambda i: (i, 0))] * 9 + [
            _full((D, D)), _full((D, D)), _full((1, D)),
            _full((D, D)), _full((1, D)), _full((1, D)), _full((1, D)),
        ],
        out_specs=pl.BlockSpec((BN, D), lambda i: (i, 0)),
        out_shape=jax.ShapeDtypeStruct((N, D), jnp.float32),
    )(x, *ps, wua, wub, bu1, wu2, bu2, g, b)


_SC_MESH = plsc.VectorSubcoreMesh(core_axis_name="c", subcore_axis_name="s",
                                  num_cores=NC, num_subcores=NS)


@functools.cache
def _make_gather(g0, gk):
    ng = (gk + NW - 1) // NW + 1

    @functools.partial(
        pl.kernel,
        out_type=jax.ShapeDtypeStruct((gk * GRP, D), jnp.float32),
        mesh=_SC_MESH,
        scratch_types=(
            [pltpu.VMEM((ng * GRP,), jnp.int32)] * 2
            + [pltpu.VMEM((GRP, D), jnp.float32)] * 6
            + [pltpu.SemaphoreType.DMA] * 9
        ),
    )
    def gather(xa_hbm, xb_hbm, dst_hbm, src_hbm, s_hbm,
               idxd_all, idxs_all, ra0, rb0, ra1, rb1, ra2, rb2,
               sa0, sb0, sa1, sb1, sa2, sb2, sw0, sw1, sw2):
        c = lax.axis_index("c")
        s = lax.axis_index("s")
        wid = s * NC + c
        gl = wid * gk // NW
        n = (wid + 1) * gk // NW - gl
        ras = (ra0, ra1, ra2)
        rbs = (rb0, rb1, rb2)
        sas = (sa0, sa1, sa2)
        sbs = (sb0, sb1, sb2)
        sws = (sw0, sw1, sw2)

        pltpu.sync_copy(dst_hbm.at[pl.ds((g0 + gl) * GRP, ng * GRP)], idxd_all)
        pltpu.sync_copy(src_hbm.at[pl.ds((g0 + gl) * GRP, ng * GRP)], idxs_all)

        def issue(k, q):
            pltpu.async_copy(xa_hbm.at[idxd_all.at[pl.ds(k * GRP, GRP)]],
                             ras[q], sas[q])
            pltpu.async_copy(xb_hbm.at[idxs_all.at[pl.ds(k * GRP, GRP)]],
                             rbs[q], sbs[q])

        def wait_gather(q):
            pltpu.make_async_copy(xa_hbm.at[pl.ds(0, GRP)], ras[q],
                                  sas[q]).wait()
            pltpu.make_async_copy(xb_hbm.at[pl.ds(0, GRP)], rbs[q],
                                  sbs[q]).wait()

        def wait_write(q):
            pltpu.make_async_copy(ras[q], s_hbm.at[pl.ds(0, GRP)],
                                  sws[q]).wait()

        def add_and_store(k, q):
            ra, rb = ras[q], rbs[q]

            def add_row(r, carry):
                for kk in range(D // 16):
                    sl = pl.ds(kk * 16, 16)
                    ra[r, sl] = ra[r, sl] + rb[r, sl]
                return carry

            lax.fori_loop(0, GRP, add_row, 0, unroll=4)
            pltpu.async_copy(ra, s_hbm.at[pl.ds((gl + k) * GRP, GRP)], sws[q])

        issue(0, 0)

        def body(k, carry):
            q = k % 3

            for qs in range(3):
                @pl.when(q == qs)
                def _(qs=qs):
                    q1 = (qs + 1) % 3

                    @pl.when((k >= 2) & (k + 1 < n))
                    def _():
                        wait_write(q1)

                    @pl.when(k + 1 < n)
                    def _():
                        issue(k + 1, q1)

                    wait_gather(qs)
                    add_and_store(k, qs)

            return carry

        lax.fori_loop(0, n, body, 0)
        wait_write(0)
        wait_write(1)
        wait_write(2)

    return gather


@functools.cache
def _make_scatter(o0, ok, gk):
    nok = ((ok + NW - 1) // NW + 1) * 8

    @functools.partial(
        pl.kernel,
        out_type=jax.ShapeDtypeStruct((NC, NPAD, D), jnp.float32),
        mesh=_SC_MESH,
        scratch_types=[
            pltpu.VMEM_SHARED((NPAD, D), jnp.float32),
            pltpu.VMEM((nok, GRP), jnp.int32),
            pltpu.VMEM((GRP, D), jnp.float32),
            pltpu.VMEM((GRP, D), jnp.float32),
            pltpu.SemaphoreType.DMA,
            pltpu.SemaphoreType.DMA,
            pltpu.SemaphoreType.DMA,
            pltpu.SemaphoreType.DMA,
        ],
    )
    def scatter(msg_hbm, dst2_hbm, zeros_hbm, parts_hbm,
                accum, idx2, rows0, rows1, seml0, seml1, sems0, sems1):
        c = lax.axis_index("c")
        s = lax.axis_index("s")
        wid = s * NC + c
        ol = wid * ok // NW
        n = ((wid + 1) * ok // NW - ol) * 8
        gl = ol * 8
        r0 = s * RPT

        pltpu.sync_copy(zeros_hbm.at[pl.ds(r0, RPT)], accum.at[pl.ds(r0, RPT)])
        pltpu.sync_copy(dst2_hbm.at[pl.ds((o0 + ol) * 8, nok)], idx2)
        plsc.subcore_barrier()

        def msg_rows(k):
            ggl = gl + k
            off = jnp.where(ggl < gk, ggl, 0) * GRP
            return msg_hbm.at[pl.ds(off, GRP)]

        def load(k, rows, sem):
            pltpu.async_copy(msg_rows(k), rows, sem)

        def wait_load(rows, sem):
            pltpu.make_async_copy(msg_hbm.at[pl.ds(0, GRP)], rows, sem).wait()

        def scat(k, rows, sem):
            pltpu.async_copy(rows, accum.at[idx2.at[k]], sem, add=True)

        def wait_scat(rows, sem):
            pltpu.make_async_copy(rows, accum.at[pl.ds(0, GRP)], sem).wait()

        load(0, rows0, seml0)

        def body(k, carry):
            even = (k % 2) == 0

            @pl.when(even)
            def _():
                @pl.when((k >= 2) & (k + 1 < n))
                def _():
                    wait_scat(rows1, sems1)

                @pl.when(k + 1 < n)
                def _():
                    load(k + 1, rows1, seml1)

                wait_load(rows0, seml0)
                scat(k, rows0, sems0)

            @pl.when(jnp.logical_not(even))
            def _():
                @pl.when(k + 1 < n)
                def _():
                    wait_scat(rows0, sems0)
                    load(k + 1, rows0, seml0)

                wait_load(rows1, seml1)
                scat(k, rows1, sems1)

            return carry

        lax.fori_loop(0, n, body, 0)
        wait_scat(rows0, sems0)
        wait_scat(rows1, sems1)
        plsc.subcore_barrier()

        pltpu.sync_copy(accum.at[pl.ds(r0, RPT)],
                        parts_hbm.at[c, pl.ds(r0, RPT)])

    return scatter


def kernel(x, edge_index, edge_attr, We1, be1, We2, be2, Wm1, bm1, Wm2, bm2,
           Wu1, bu1, Wu2, bu2, Wg, bg, ln_gamma, ln_beta):
    src = edge_index[0]
    dst = edge_index[1]

    Wm1a, Wm1b, Wm1c = Wm1[:D], Wm1[D:2 * D], Wm1[2 * D:]
    Wem = We2 @ Wm1c
    b_m = (be2 @ Wm1c + bm1).reshape(1, D)
    Wge = We2 @ Wg
    b_g = (be2 @ Wg + bg).reshape(1, D)

    xa, xb = _node_pre(x, Wm1a, Wm1b, b_m)

    pad1 = jnp.zeros((256,), jnp.int32)
    dst_f = jnp.concatenate([dst, pad1])
    src_f = jnp.concatenate([src, pad1])
    dst2p = jnp.concatenate(
        [dst.reshape(G, GRP),
         jnp.full((GP2 - G, GRP), JUNK_ROW, jnp.int32)], axis=0)
    zeros = jnp.zeros((NPAD, D), jnp.float32)

    be1r = be1.reshape(1, D)
    bm2r = bm2.reshape(1, D)

    parts = []
    for (g0, gk), (o0, ok) in zip(CHUNKS, OCT_CHUNKS):
        s_k = _make_gather(g0, gk)(xa, xb, dst_f, src_f)
        msg_k = _edge_mlp(edge_attr, s_k, We1, be1r, Wem, Wge, b_g,
                          Wm2, bm2r, g0 * GRP)
        parts.append(_make_scatter(o0, ok, gk)(msg_k, dst2p, zeros))

    Wu1a, Wu1b = Wu1[:D], Wu1[D:]
    ps = [p[i, :N] for p in parts for i in range(NC)]
    return _node_upd(x, ps, Wu1a, Wu1b, bu1.reshape(1, D), Wu2,
                     bu2.reshape(1, D), ln_gamma.reshape(1, D),
                     ln_beta.reshape(1, D))


# speedup vs baseline: 1.0346x; 1.0346x over previous
import functools

import jax
import jax.numpy as jnp
from jax import lax
from jax.experimental import pallas as pl
from jax.experimental.pallas import tpu as pltpu
from jax.experimental.pallas import tpu_sc as plsc

N = 10000
E = 320000
D = 128

BN = 2000
BE = 2560

NC = 2
NS = 16
NW = NC * NS
GRP = 128
G = E // GRP
NPAD = 10240
RPT = NPAD // NS
JUNK_ROW = NPAD - 1

CHUNKS = ((0, 800), (800, 1700))
GP2 = 2512
OCT_CHUNKS = ((0, 100), (100, 213))


def _silu(v):
    return v * jax.nn.sigmoid(v)


def _node_pre_body(x_ref, wa_ref, wb_ref, bm_ref, xa_ref, xb_ref):
    x = x_ref[...]
    xa_ref[...] = x @ wa_ref[...] + bm_ref[...]
    xb_ref[...] = x @ wb_ref[...]


def _edge_body(ea_ref, s_ref, we1_ref, be1_ref, wem_ref, wge_ref,
               bg_ref, wm2_ref, bm2_ref, msg_ref):
    h_e = _silu(ea_ref[...] @ we1_ref[...] + be1_ref[...])
    gate = jax.nn.sigmoid(h_e @ wge_ref[...] + bg_ref[...])
    pre = h_e @ wem_ref[...] + s_ref[...]
    msg_ref[...] = (_silu(pre) @ wm2_ref[...] + bm2_ref[...]) * gate


def _node_upd_body(x_ref, p0_ref, p1_ref, p2_ref, p3_ref, wua_ref, wub_ref,
                   bu1_ref, wu2_ref, bu2_ref, g_ref, b_ref, out_ref):
    x = x_ref[...]
    aggr = (p0_ref[...] + p1_ref[...]) + (p2_ref[...] + p3_ref[...])
    u1 = _silu(aggr @ wua_ref[...] + x @ wub_ref[...] + bu1_ref[...])
    upd = u1 @ wu2_ref[...] + bu2_ref[...]
    h = x + upd
    mu = jnp.mean(h, axis=-1, keepdims=True)
    var = jnp.mean((h - mu) ** 2, axis=-1, keepdims=True)
    out_ref[...] = (h - mu) * jax.lax.rsqrt(var + 1e-5) * g_ref[...] + b_ref[...]


def _full(shape):
    return pl.BlockSpec(shape, lambda i: tuple(0 for _ in shape))


def _node_pre(x, wa, wb, bm):
    return pl.pallas_call(
        _node_pre_body,
        grid=(N // BN,),
        in_specs=[
            pl.BlockSpec((BN, D), lambda i: (i, 0)),
            _full((D, D)), _full((D, D)), _full((1, D)),
        ],
        out_specs=[pl.BlockSpec((BN, D), lambda i: (i, 0)),
                   pl.BlockSpec((BN, D), lambda i: (i, 0))],
        out_shape=[jax.ShapeDtypeStruct((N, D), jnp.float32),
                   jax.ShapeDtypeStruct((N, D), jnp.float32)],
    )(x, wa, wb, bm)


def _edge_mlp(ea, s_k, we1, be1, wem, wge, bg2, wm2, bm2, e0):
    R = ea.shape[-1]
    ek = s_k.shape[0]
    blk0 = e0 // BE
    return pl.pallas_call(
        _edge_body,
        grid=(ek // BE,),
        in_specs=[
            pl.BlockSpec((BE, R), lambda i: (i + blk0, 0)),
            pl.BlockSpec((BE, D), lambda i: (i, 0)),
            _full((R, D)), _full((1, D)), _full((D, D)), _full((D, D)),
            _full((1, D)), _full((D, D)), _full((1, D)),
        ],
        out_specs=pl.BlockSpec((BE, D), lambda i: (i, 0)),
        out_shape=jax.ShapeDtypeStruct((ek, D), jnp.float32),
    )(ea, s_k, we1, be1, wem, wge, bg2, wm2, bm2)


def _node_upd(x, ps, wua, wub, bu1, wu2, bu2, g, b):
    return pl.pallas_call(
        _node_upd_body,
        grid=(N // BN,),
        in_specs=[pl.BlockSpec((BN, D), lambda i: (i, 0))] * 5 + [
            _full((D, D)), _full((D, D)), _full((1, D)),
            _full((D, D)), _full((1, D)), _full((1, D)), _full((1, D)),
        ],
        out_specs=pl.BlockSpec((BN, D), lambda i: (i, 0)),
        out_shape=jax.ShapeDtypeStruct((N, D), jnp.float32),
    )(x, *ps, wua, wub, bu1, wu2, bu2, g, b)


_SC_MESH = plsc.VectorSubcoreMesh(core_axis_name="c", subcore_axis_name="s",
                                  num_cores=NC, num_subcores=NS)


@functools.cache
def _make_gather(g0, gk):
    ng = (gk + NW - 1) // NW + 1

    @functools.partial(
        pl.kernel,
        out_type=jax.ShapeDtypeStruct((gk * GRP, D), jnp.float32),
        mesh=_SC_MESH,
        scratch_types=(
            [pltpu.VMEM((ng * GRP,), jnp.int32)] * 2
            + [pltpu.VMEM((GRP, D), jnp.float32)] * 6
            + [pltpu.SemaphoreType.DMA] * 9
        ),
    )
    def gather(xa_hbm, xb_hbm, dst_hbm, src_hbm, s_hbm,
               idxd_all, idxs_all, ra0, rb0, ra1, rb1, ra2, rb2,
               sa0, sb0, sa1, sb1, sa2, sb2, sw0, sw1, sw2):
        c = lax.axis_index("c")
        s = lax.axis_index("s")
        wid = s * NC + c
        gl = wid * gk // NW
        n = (wid + 1) * gk // NW - gl
        ras = (ra0, ra1, ra2)
        rbs = (rb0, rb1, rb2)
        sas = (sa0, sa1, sa2)
        sbs = (sb0, sb1, sb2)
        sws = (sw0, sw1, sw2)

        pltpu.sync_copy(dst_hbm.at[pl.ds((g0 + gl) * GRP, ng * GRP)], idxd_all)
        pltpu.sync_copy(src_hbm.at[pl.ds((g0 + gl) * GRP, ng * GRP)], idxs_all)

        def issue(k, q):
            pltpu.async_copy(xa_hbm.at[idxd_all.at[pl.ds(k * GRP, GRP)]],
                             ras[q], sas[q])
            pltpu.async_copy(xb_hbm.at[idxs_all.at[pl.ds(k * GRP, GRP)]],
                             rbs[q], sbs[q])

        def wait_gather(q):
            pltpu.make_async_copy(xa_hbm.at[pl.ds(0, GRP)], ras[q],
                                  sas[q]).wait()
            pltpu.make_async_copy(xb_hbm.at[pl.ds(0, GRP)], rbs[q],
                                  sbs[q]).wait()

        def wait_write(q):
            pltpu.make_async_copy(ras[q], s_hbm.at[pl.ds(0, GRP)],
                                  sws[q]).wait()

        def add_and_store(k, q):
            ra, rb = ras[q], rbs[q]

            def add_row(r, carry):
                for kk in range(D // 16):
                    sl = pl.ds(kk * 16, 16)
                    ra[r, sl] = ra[r, sl] + rb[r, sl]
                return carry

            lax.fori_loop(0, GRP, add_row, 0, unroll=4)
            pltpu.async_copy(ra, s_hbm.at[pl.ds((gl + k) * GRP, GRP)], sws[q])

        issue(0, 0)

        def body(k, carry):
            q = k % 3

            for qs in range(3):
                @pl.when(q == qs)
                def _(qs=qs):
                    q1 = (qs + 1) % 3

                    @pl.when((k >= 2) & (k + 1 < n))
                    def _():
                        wait_write(q1)

                    @pl.when(k + 1 < n)
                    def _():
                        issue(k + 1, q1)

                    wait_gather(qs)
                    add_and_store(k, qs)

            return carry

        lax.fori_loop(0, n, body, 0)
        wait_write(0)
        wait_write(1)
        wait_write(2)

    return gather


@functools.cache
def _make_scatter(o0, ok, gk):
    nok = ((ok + NW - 1) // NW + 1) * 8

    @functools.partial(
        pl.kernel,
        out_type=jax.ShapeDtypeStruct((NC, NPAD, D), jnp.float32),
        mesh=_SC_MESH,
        scratch_types=[
            pltpu.VMEM_SHARED((NPAD, D), jnp.float32),
            pltpu.VMEM((nok, GRP), jnp.int32),
            pltpu.VMEM((GRP, D), jnp.float32),
            pltpu.VMEM((GRP, D), jnp.float32),
            pltpu.SemaphoreType.DMA,
            pltpu.SemaphoreType.DMA,
            pltpu.SemaphoreType.DMA,
            pltpu.SemaphoreType.DMA,
        ],
    )
    def scatter(msg_hbm, dst2_hbm, zeros_hbm, parts_hbm,
                accum, idx2, rows0, rows1, seml0, seml1, sems0, sems1):
        c = lax.axis_index("c")
        s = lax.axis_index("s")
        wid = s * NC + c
        ol = wid * ok // NW
        n = ((wid + 1) * ok // NW - ol) * 8
        gl = ol * 8
        r0 = s * RPT

        pltpu.sync_copy(zeros_hbm.at[pl.ds(r0, RPT)], accum.at[pl.ds(r0, RPT)])
        pltpu.sync_copy(dst2_hbm.at[pl.ds((o0 + ol) * 8, nok)], idx2)
        plsc.subcore_barrier()

        def msg_rows(k):
            ggl = gl + k
            off = jnp.where(ggl < gk, ggl, 0) * GRP
            return msg_hbm.at[pl.ds(off, GRP)]

        def load(k, rows, sem):
            pltpu.async_copy(msg_rows(k), rows, sem)

        def wait_load(rows, sem):
            pltpu.make_async_copy(msg_hbm.at[pl.ds(0, GRP)], rows, sem).wait()

        def scat(k, rows, sem):
            pltpu.async_copy(rows, accum.at[idx2.at[k]], sem, add=True)

        def wait_scat(rows, sem):
            pltpu.make_async_copy(rows, accum.at[pl.ds(0, GRP)], sem).wait()

        load(0, rows0, seml0)

        def body(k, carry):
            even = (k % 2) == 0

            @pl.when(even)
            def _():
                @pl.when((k >= 2) & (k + 1 < n))
                def _():
                    wait_scat(rows1, sems1)

                @pl.when(k + 1 < n)
                def _():
                    load(k + 1, rows1, seml1)

                wait_load(rows0, seml0)
                scat(k, rows0, sems0)

            @pl.when(jnp.logical_not(even))
            def _():
                @pl.when(k + 1 < n)
                def _():
                    wait_scat(rows0, sems0)
                    load(k + 1, rows0, seml0)

                wait_load(rows1, seml1)
                scat(k, rows1, sems1)

            return carry

        lax.fori_loop(0, n, body, 0)
        wait_scat(rows0, sems0)
        wait_scat(rows1, sems1)
        plsc.subcore_barrier()

        pltpu.sync_copy(accum.at[pl.ds(r0, RPT)],
                        parts_hbm.at[c, pl.ds(r0, RPT)])

    return scatter


def kernel(x, edge_index, edge_attr, We1, be1, We2, be2, Wm1, bm1, Wm2, bm2,
           Wu1, bu1, Wu2, bu2, Wg, bg, ln_gamma, ln_beta):
    src = edge_index[0]
    dst = edge_index[1]

    Wm1a, Wm1b, Wm1c = Wm1[:D], Wm1[D:2 * D], Wm1[2 * D:]
    Wem = We2 @ Wm1c
    b_m = (be2 @ Wm1c + bm1).reshape(1, D)
    Wge = We2 @ Wg
    b_g = (be2 @ Wg + bg).reshape(1, D)

    xa, xb = _node_pre(x, Wm1a, Wm1b, b_m)

    pad1 = jnp.zeros((256,), jnp.int32)
    dst_f = jnp.concatenate([dst, pad1])
    src_f = jnp.concatenate([src, pad1])
    dst2p = jnp.concatenate(
        [dst.reshape(G, GRP),
         jnp.full((GP2 - G, GRP), JUNK_ROW, jnp.int32)], axis=0)
    zeros = jnp.zeros((NPAD, D), jnp.float32)

    be1r = be1.reshape(1, D)
    bm2r = bm2.reshape(1, D)

    parts = []
    for (g0, gk), (o0, ok) in zip(CHUNKS, OCT_CHUNKS):
        s_k = _make_gather(g0, gk)(xa, xb, dst_f, src_f)
        msg_k = _edge_mlp(edge_attr, s_k, We1, be1r, Wem, Wge, b_g,
                          Wm2, bm2r, g0 * GRP)
        parts.append(_make_scatter(o0, ok, gk)(msg_k, dst2p, zeros))

    Wu1a, Wu1b = Wu1[:D], Wu1[D:]
    ps = [p[i, :N] for p in parts for i in range(NC)]
    return _node_upd(x, ps, Wu1a, Wu1b, bu1.reshape(1, D), Wu2,
                     bu2.reshape(1, D), ln_gamma.reshape(1, D),
                     ln_beta.reshape(1, D))


# speedup vs baseline: 1.0921x; 1.0555x over previous
import functools

import jax
import jax.numpy as jnp
from jax import lax
from jax.experimental import pallas as pl
from jax.experimental.pallas import tpu as pltpu
from jax.experimental.pallas import tpu_sc as plsc

N = 10000
E = 320000
D = 128

BN = 2000
BE = 2560

NC = 2
NS = 16
NW = NC * NS
GRP = 128
G = E // GRP
NPAD = 10240
RPT = NPAD // NS
JUNK_ROW = NPAD - 1

CHUNKS = ((0, 1480), (1480, 1020))
GP2 = 2512
OCT_CHUNKS = ((0, 185), (185, 128))


def _silu(v):
    return v * jax.nn.sigmoid(v)


def _node_pre_body(x_ref, wa_ref, wb_ref, bm_ref, xa_ref, xb_ref):
    x = x_ref[...]
    xa_ref[...] = x @ wa_ref[...] + bm_ref[...]
    xb_ref[...] = x @ wb_ref[...]


def _edge_body(ea_ref, s_ref, we1_ref, be1_ref, wem_ref, wge_ref,
               bg_ref, wm2_ref, bm2_ref, msg_ref):
    h_e = _silu(ea_ref[...] @ we1_ref[...] + be1_ref[...])
    gate = jax.nn.sigmoid(h_e @ wge_ref[...] + bg_ref[...])
    pre = h_e @ wem_ref[...] + s_ref[...]
    msg_ref[...] = (_silu(pre) @ wm2_ref[...] + bm2_ref[...]) * gate


def _node_upd_body(x_ref, p0_ref, p1_ref, p2_ref, p3_ref, wua_ref, wub_ref,
                   bu1_ref, wu2_ref, bu2_ref, g_ref, b_ref, out_ref):
    x = x_ref[...]
    aggr = (p0_ref[...] + p1_ref[...]) + (p2_ref[...] + p3_ref[...])
    u1 = _silu(aggr @ wua_ref[...] + x @ wub_ref[...] + bu1_ref[...])
    upd = u1 @ wu2_ref[...] + bu2_ref[...]
    h = x + upd
    mu = jnp.mean(h, axis=-1, keepdims=True)
    var = jnp.mean((h - mu) ** 2, axis=-1, keepdims=True)
    out_ref[...] = (h - mu) * jax.lax.rsqrt(var + 1e-5) * g_ref[...] + b_ref[...]


def _full(shape):
    return pl.BlockSpec(shape, lambda i: tuple(0 for _ in shape))


def _node_pre(x, wa, wb, bm):
    return pl.pallas_call(
        _node_pre_body,
        grid=(N // BN,),
        in_specs=[
            pl.BlockSpec((BN, D), lambda i: (i, 0)),
            _full((D, D)), _full((D, D)), _full((1, D)),
        ],
        out_specs=[pl.BlockSpec((BN, D), lambda i: (i, 0)),
                   pl.BlockSpec((BN, D), lambda i: (i, 0))],
        out_shape=[jax.ShapeDtypeStruct((N, D), jnp.float32),
                   jax.ShapeDtypeStruct((N, D), jnp.float32)],
    )(x, wa, wb, bm)


def _edge_mlp(ea, s_k, we1, be1, wem, wge, bg2, wm2, bm2, e0):
    R = ea.shape[-1]
    ek = s_k.shape[0]
    blk0 = e0 // BE
    return pl.pallas_call(
        _edge_body,
        grid=(ek // BE,),
        in_specs=[
            pl.BlockSpec((BE, R), lambda i: (i + blk0, 0)),
            pl.BlockSpec((BE, D), lambda i: (i, 0)),
            _full((R, D)), _full((1, D)), _full((D, D)), _full((D, D)),
            _full((1, D)), _full((D, D)), _full((1, D)),
        ],
        out_specs=pl.BlockSpec((BE, D), lambda i: (i, 0)),
        out_shape=jax.ShapeDtypeStruct((ek, D), jnp.float32),
    )(ea, s_k, we1, be1, wem, wge, bg2, wm2, bm2)


def _node_upd(x, ps, wua, wub, bu1, wu2, bu2, g, b):
    return pl.pallas_call(
        _node_upd_body,
        grid=(N // BN,),
        in_specs=[pl.BlockSpec((BN, D), lambda i: (i, 0))] * 5 + [
            _full((D, D)), _full((D, D)), _full((1, D)),
            _full((D, D)), _full((1, D)), _full((1, D)), _full((1, D)),
        ],
        out_specs=pl.BlockSpec((BN, D), lambda i: (i, 0)),
        out_shape=jax.ShapeDtypeStruct((N, D), jnp.float32),
    )(x, *ps, wua, wub, bu1, wu2, bu2, g, b)


_SC_MESH = plsc.VectorSubcoreMesh(core_axis_name="c", subcore_axis_name="s",
                                  num_cores=NC, num_subcores=NS)


@functools.cache
def _make_gather(g0, gk):
    ng = (gk + NW - 1) // NW + 1

    @functools.partial(
        pl.kernel,
        out_type=jax.ShapeDtypeStruct((gk * GRP, D), jnp.float32),
        mesh=_SC_MESH,
        scratch_types=(
            [pltpu.VMEM((ng * GRP,), jnp.int32)] * 2
            + [pltpu.VMEM((GRP, D), jnp.float32)] * 6
            + [pltpu.SemaphoreType.DMA] * 9
        ),
    )
    def gather(xa_hbm, xb_hbm, dst_hbm, src_hbm, s_hbm,
               idxd_all, idxs_all, ra0, rb0, ra1, rb1, ra2, rb2,
               sa0, sb0, sa1, sb1, sa2, sb2, sw0, sw1, sw2):
        c = lax.axis_index("c")
        s = lax.axis_index("s")
        wid = s * NC + c
        gl = wid * gk // NW
        n = (wid + 1) * gk // NW - gl
        ras = (ra0, ra1, ra2)
        rbs = (rb0, rb1, rb2)
        sas = (sa0, sa1, sa2)
        sbs = (sb0, sb1, sb2)
        sws = (sw0, sw1, sw2)

        pltpu.sync_copy(dst_hbm.at[pl.ds((g0 + gl) * GRP, ng * GRP)], idxd_all)
        pltpu.sync_copy(src_hbm.at[pl.ds((g0 + gl) * GRP, ng * GRP)], idxs_all)

        def issue(k, q):
            pltpu.async_copy(xa_hbm.at[idxd_all.at[pl.ds(k * GRP, GRP)]],
                             ras[q], sas[q])
            pltpu.async_copy(xb_hbm.at[idxs_all.at[pl.ds(k * GRP, GRP)]],
                             rbs[q], sbs[q])

        def wait_gather(q):
            pltpu.make_async_copy(xa_hbm.at[pl.ds(0, GRP)], ras[q],
                                  sas[q]).wait()
            pltpu.make_async_copy(xb_hbm.at[pl.ds(0, GRP)], rbs[q],
                                  sbs[q]).wait()

        def wait_write(q):
            pltpu.make_async_copy(ras[q], s_hbm.at[pl.ds(0, GRP)],
                                  sws[q]).wait()

        def add_and_store(k, q):
            ra, rb = ras[q], rbs[q]

            def add_row(r, carry):
                for kk in range(D // 16):
                    sl = pl.ds(kk * 16, 16)
                    ra[r, sl] = ra[r, sl] + rb[r, sl]
                return carry

            lax.fori_loop(0, GRP, add_row, 0, unroll=4)
            pltpu.async_copy(ra, s_hbm.at[pl.ds((gl + k) * GRP, GRP)], sws[q])

        issue(0, 0)

        def body(k, carry):
            q = k % 3

            for qs in range(3):
                @pl.when(q == qs)
                def _(qs=qs):
                    q1 = (qs + 1) % 3

                    @pl.when((k >= 2) & (k + 1 < n))
                    def _():
                        wait_write(q1)

                    @pl.when(k + 1 < n)
                    def _():
                        issue(k + 1, q1)

                    wait_gather(qs)
                    add_and_store(k, qs)

            return carry

        lax.fori_loop(0, n, body, 0)
        wait_write(0)
        wait_write(1)
        wait_write(2)

    return gather


@functools.cache
def _make_scatter(o0, ok, gk):
    nok = ((ok + NW - 1) // NW + 1) * 8

    @functools.partial(
        pl.kernel,
        out_type=jax.ShapeDtypeStruct((NC, NPAD, D), jnp.float32),
        mesh=_SC_MESH,
        scratch_types=[
            pltpu.VMEM_SHARED((NPAD, D), jnp.float32),
            pltpu.VMEM((nok, GRP), jnp.int32),
            pltpu.VMEM((GRP, D), jnp.float32),
            pltpu.VMEM((GRP, D), jnp.float32),
            pltpu.SemaphoreType.DMA,
            pltpu.SemaphoreType.DMA,
            pltpu.SemaphoreType.DMA,
            pltpu.SemaphoreType.DMA,
        ],
    )
    def scatter(msg_hbm, dst2_hbm, zeros_hbm, parts_hbm,
                accum, idx2, rows0, rows1, seml0, seml1, sems0, sems1):
        c = lax.axis_index("c")
        s = lax.axis_index("s")
        wid = s * NC + c
        ol = wid * ok // NW
        n = ((wid + 1) * ok // NW - ol) * 8
        gl = ol * 8
        r0 = s * RPT

        pltpu.sync_copy(zeros_hbm.at[pl.ds(r0, RPT)], accum.at[pl.ds(r0, RPT)])
        pltpu.sync_copy(dst2_hbm.at[pl.ds((o0 + ol) * 8, nok)], idx2)
        plsc.subcore_barrier()

        def msg_rows(k):
            ggl = gl + k
            off = jnp.where(ggl < gk, ggl, 0) * GRP
            return msg_hbm.at[pl.ds(off, GRP)]

        def load(k, rows, sem):
            pltpu.async_copy(msg_rows(k), rows, sem)

        def wait_load(rows, sem):
            pltpu.make_async_copy(msg_hbm.at[pl.ds(0, GRP)], rows, sem).wait()

        def scat(k, rows, sem):
            pltpu.async_copy(rows, accum.at[idx2.at[k]], sem, add=True)

        def wait_scat(rows, sem):
            pltpu.make_async_copy(rows, accum.at[pl.ds(0, GRP)], sem).wait()

        load(0, rows0, seml0)

        def body(k, carry):
            even = (k % 2) == 0

            @pl.when(even)
            def _():
                @pl.when((k >= 2) & (k + 1 < n))
                def _():
                    wait_scat(rows1, sems1)

                @pl.when(k + 1 < n)
                def _():
                    load(k + 1, rows1, seml1)

                wait_load(rows0, seml0)
                scat(k, rows0, sems0)

            @pl.when(jnp.logical_not(even))
            def _():
                @pl.when(k + 1 < n)
                def _():
                    wait_scat(rows0, sems0)
                    load(k + 1, rows0, seml0)

                wait_load(rows1, seml1)
                scat(k, rows1, sems1)

            return carry

        lax.fori_loop(0, n, body, 0)
        wait_scat(rows0, sems0)
        wait_scat(rows1, sems1)
        plsc.subcore_barrier()

        pltpu.sync_copy(accum.at[pl.ds(r0, RPT)],
                        parts_hbm.at[c, pl.ds(r0, RPT)])

    return scatter


def kernel(x, edge_index, edge_attr, We1, be1, We2, be2, Wm1, bm1, Wm2, bm2,
           Wu1, bu1, Wu2, bu2, Wg, bg, ln_gamma, ln_beta):
    src = edge_index[0]
    dst = edge_index[1]

    Wm1a, Wm1b, Wm1c = Wm1[:D], Wm1[D:2 * D], Wm1[2 * D:]
    Wem = We2 @ Wm1c
    b_m = (be2 @ Wm1c + bm1).reshape(1, D)
    Wge = We2 @ Wg
    b_g = (be2 @ Wg + bg).reshape(1, D)

    xa, xb = _node_pre(x, Wm1a, Wm1b, b_m)

    pad1 = jnp.zeros((256,), jnp.int32)
    dst_f = jnp.concatenate([dst, pad1])
    src_f = jnp.concatenate([src, pad1])
    dst2p = jnp.concatenate(
        [dst.reshape(G, GRP),
         jnp.full((GP2 - G, GRP), JUNK_ROW, jnp.int32)], axis=0)
    zeros = jnp.zeros((NPAD, D), jnp.float32)

    be1r = be1.reshape(1, D)
    bm2r = bm2.reshape(1, D)

    parts = []
    for (g0, gk), (o0, ok) in zip(CHUNKS, OCT_CHUNKS):
        s_k = _make_gather(g0, gk)(xa, xb, dst_f, src_f)
        msg_k = _edge_mlp(edge_attr, s_k, We1, be1r, Wem, Wge, b_g,
                          Wm2, bm2r, g0 * GRP)
        parts.append(_make_scatter(o0, ok, gk)(msg_k, dst2p, zeros))

    Wu1a, Wu1b = Wu1[:D], Wu1[D:]
    ps = [p[i, :N] for p in parts for i in range(NC)]
    return _node_upd(x, ps, Wu1a, Wu1b, bu1.reshape(1, D), Wu2,
                     bu2.reshape(1, D), ln_gamma.reshape(1, D),
                     ln_beta.reshape(1, D))
